# baseline (speedup 1.0000x reference)
import functools

import jax
import jax.numpy as jnp
from jax import lax
from jax.experimental import pallas as pl
from jax.experimental.pallas import tpu as pltpu
from jax.experimental.pallas import tpu_sc as plsc

N_NODES = 100000
N_EDGES = 6400000
IN_DIM = 8
HIDDEN = 128
XCOLS = 16

NUM_WORKERS = 32
ROW = 128
ROWS_PER_ITER = 2
EDGE_ROWS = N_EDGES // ROW
N_CHUNKS = EDGE_ROWS // ROWS_PER_ITER
CHUNKS_MAIN = -(-N_CHUNKS // NUM_WORKERS)
CHUNKS_LAST = N_CHUNKS - 31 * CHUNKS_MAIN
AGG_ROWS = 102400
ZROWS = AGG_ROWS // 16


def _sc_body(xaug_hbm, ei_hbm, zeros_hbm, out_hbm,
             idx_v0, idx_v1, rows_v0, rows_v1, agg_sh,
             gsem0, gsem1, ssem0, ssem1):
    c = lax.axis_index("c")
    s = lax.axis_index("s")
    w = c * 16 + s

    pltpu.sync_copy(zeros_hbm, agg_sh.at[pl.ds(s * ZROWS, ZROWS)])
    plsc.subcore_barrier()

    base_chunk = w * CHUNKS_MAIN
    n = jnp.where(w == NUM_WORKERS - 1, CHUNKS_LAST, CHUNKS_MAIN)

    def load_idx(i, idx_v):
        r0 = (base_chunk + i) * ROWS_PER_ITER
        pltpu.sync_copy(ei_hbm.at[:, pl.ds(r0, ROWS_PER_ITER)], idx_v)

    def fire_gathers(idx_v, rows_v, gsem):
        for j in range(ROWS_PER_ITER):
            pltpu.async_copy(xaug_hbm.at[idx_v.at[1].at[j]],
                             rows_v.at[pl.ds(j * ROW, ROW)], gsem)

    def drain_gathers(idx_v, rows_v, gsem):
        for j in range(ROWS_PER_ITER):
            pltpu.make_async_copy(xaug_hbm.at[idx_v.at[1].at[j]],
                                  rows_v.at[pl.ds(j * ROW, ROW)], gsem).wait()

    def fire_scatters(idx_v, rows_v, ssem):
        for j in range(ROWS_PER_ITER):
            pltpu.async_copy(rows_v.at[pl.ds(j * ROW, ROW)],
                             agg_sh.at[idx_v.at[0].at[j]], ssem, add=True)

    def drain_scatters(idx_v, rows_v, ssem):
        for j in range(ROWS_PER_ITER):
            pltpu.make_async_copy(rows_v.at[pl.ds(j * ROW, ROW)],
                                  agg_sh.at[idx_v.at[0].at[j]], ssem).wait()

    bufs = ((idx_v0, rows_v0, gsem0, ssem0),
            (idx_v1, rows_v1, gsem1, ssem1))

    load_idx(0, idx_v0)
    fire_gathers(idx_v0, rows_v0, gsem0)

    def stage(i, cur, nxt):
        idx_c, rows_c, gsem_c, ssem_c = cur
        idx_n, rows_n, gsem_n, ssem_n = nxt

        @pl.when(i >= 1)
        def _():
            drain_scatters(idx_n, rows_n, ssem_n)

        @pl.when(i + 1 < n)
        def _():
            load_idx(i + 1, idx_n)

        drain_gathers(idx_c, rows_c, gsem_c)

        @pl.when(i + 1 < n)
        def _():
            fire_gathers(idx_n, rows_n, gsem_n)

        fire_scatters(idx_c, rows_c, ssem_c)

    @pl.loop(0, n)
    def _edge_iter(i):
        even = (i % 2) == 0

        @pl.when(even)
        def _():
            stage(i, bufs[0], bufs[1])

        @pl.when(jnp.logical_not(even))
        def _():
            stage(i, bufs[1], bufs[0])

    last_even = ((n - 1) % 2) == 0

    @pl.when(last_even)
    def _():
        drain_scatters(idx_v0, rows_v0, ssem0)

    @pl.when(jnp.logical_not(last_even))
    def _():
        drain_scatters(idx_v1, rows_v1, ssem1)

    plsc.subcore_barrier()
    pltpu.sync_copy(agg_sh.at[pl.ds(s * ZROWS, ZROWS)],
                    out_hbm.at[c].at[pl.ds(s * ZROWS, ZROWS)])


def _scatter_parts(xaug, ei3, zeros_hbm):
    mesh = plsc.VectorSubcoreMesh(core_axis_name="c", subcore_axis_name="s")
    f = pl.kernel(
        _sc_body,
        out_type=jax.ShapeDtypeStruct((2, AGG_ROWS, XCOLS), jnp.float32),
        mesh=mesh,
        scratch_types=[
            pltpu.VMEM((2, ROWS_PER_ITER, ROW), jnp.int32),
            pltpu.VMEM((2, ROWS_PER_ITER, ROW), jnp.int32),
            pltpu.VMEM((ROWS_PER_ITER * ROW, XCOLS), jnp.float32),
            pltpu.VMEM((ROWS_PER_ITER * ROW, XCOLS), jnp.float32),
            pltpu.VMEM_SHARED((AGG_ROWS, XCOLS), jnp.float32),
            pltpu.SemaphoreType.DMA,
            pltpu.SemaphoreType.DMA,
            pltpu.SemaphoreType.DMA,
            pltpu.SemaphoreType.DMA,
        ],
        compiler_params=pltpu.CompilerParams(use_tc_tiling_on_sc=False),
    )
    return f(xaug, ei3, zeros_hbm)


NBLK = 50
BLK = N_NODES // NBLK


def _mlp_body(parts_ref, x_ref, w1t_ref, w1p_ref, b1_ref, w2t_ref, b2_ref,
              w3t_ref, b3_ref, wv_ref, bv_ref, out_ref, acc_ref):
    i = pl.program_id(0)
    p = parts_ref[0] + parts_ref[1]
    lane = lax.broadcasted_iota(jnp.int32, (BLK, XCOLS), 1)
    deg = jnp.sum(jnp.where(lane == IN_DIM, p, 0.0), axis=1, keepdims=True)
    deg = jnp.maximum(deg, 1.0)
    aggw = lax.dot_general(p, w1p_ref[...],
                           (((1,), (0,)), ((), ()))) / deg
    xw = lax.dot_general(x_ref[...], w1t_ref[...], (((1,), (0,)), ((), ())))
    h = jnp.maximum(xw + aggw + b1_ref[...], 0.0)
    h = jnp.maximum(lax.dot_general(h, w2t_ref[...], (((1,), (0,)), ((), ())))
                    + b2_ref[...], 0.0)
    h = jnp.maximum(lax.dot_general(h, w3t_ref[...], (((1,), (0,)), ((), ())))
                    + b3_ref[...], 0.0)
    part_sum = jnp.sum(h, axis=0, keepdims=True)

    @pl.when(i == 0)
    def _():
        acc_ref[...] = part_sum

    @pl.when(i > 0)
    def _():
        acc_ref[...] = acc_ref[...] + part_sum

    @pl.when(i == NBLK - 1)
    def _():
        m = acc_ref[...] / jnp.float32(N_NODES)
        v = jnp.sum(m * wv_ref[...], axis=1, keepdims=True) + bv_ref[...]
        out_ref[...] = jnp.tanh(v)


def _mlp(parts, x, w1t, w1p, b1, w2t, b2, w3t, b3, wv, bv):
    return pl.pallas_call(
        _mlp_body,
        grid=(NBLK,),
        in_specs=[
            pl.BlockSpec((2, BLK, XCOLS), lambda i: (0, i, 0)),
            pl.BlockSpec((BLK, IN_DIM), lambda i: (i, 0)),
            pl.BlockSpec((IN_DIM, HIDDEN), lambda i: (0, 0)),
            pl.BlockSpec((XCOLS, HIDDEN), lambda i: (0, 0)),
            pl.BlockSpec((1, HIDDEN), lambda i: (0, 0)),
            pl.BlockSpec((HIDDEN, HIDDEN), lambda i: (0, 0)),
            pl.BlockSpec((1, HIDDEN), lambda i: (0, 0)),
            pl.BlockSpec((HIDDEN, HIDDEN), lambda i: (0, 0)),
            pl.BlockSpec((1, HIDDEN), lambda i: (0, 0)),
            pl.BlockSpec((1, HIDDEN), lambda i: (0, 0)),
            pl.BlockSpec((1, 1), lambda i: (0, 0)),
        ],
        out_specs=pl.BlockSpec((1, 1), lambda i: (0, 0)),
        out_shape=jax.ShapeDtypeStruct((1, 1), jnp.float32),
        scratch_shapes=[pltpu.VMEM((1, HIDDEN), jnp.float32)],
        compiler_params=pltpu.CompilerParams(
            dimension_semantics=("arbitrary",)),
    )(parts, x, w1t, w1p, b1, w2t, b2, w3t, b3, wv, bv)


def kernel(x, edge_index, W1, b1, W2, b2, W3, b3, Wv, bv):
    ei3 = edge_index.reshape(2, EDGE_ROWS, ROW)
    xaug = jnp.zeros((N_NODES, XCOLS), dtype=jnp.float32)
    xaug = xaug.at[:, 0:IN_DIM].set(x).at[:, IN_DIM].set(1.0)
    zeros_hbm = jnp.zeros((ZROWS, XCOLS), dtype=jnp.float32)

    parts = _scatter_parts(xaug, ei3, zeros_hbm)

    w1p = jnp.zeros((XCOLS, HIDDEN), dtype=jnp.float32).at[0:IN_DIM].set(W1.T)
    out = _mlp(parts, x, W1.T, w1p, b1.reshape(1, -1), W2.T,
               b2.reshape(1, -1), W3.T, b3.reshape(1, -1), Wv.reshape(1, -1),
               bv.reshape(1, 1))
    return jnp.squeeze(out)


# speedup vs baseline: 1.2360x; 1.2360x over previous
import functools

import jax
import jax.numpy as jnp
from jax import lax
from jax.experimental import pallas as pl
from jax.experimental.pallas import tpu as pltpu
from jax.experimental.pallas import tpu_sc as plsc

N_NODES = 100000
N_EDGES = 6400000
IN_DIM = 8
HIDDEN = 128
XCOLS = 16

NUM_WORKERS = 32
ROW = 128
ROWS_PER_ITER = 4
EDGE_ROWS = N_EDGES // ROW
N_CHUNKS = EDGE_ROWS // ROWS_PER_ITER
CHUNKS_MAIN = -(-N_CHUNKS // NUM_WORKERS)
CHUNKS_LAST = N_CHUNKS - 31 * CHUNKS_MAIN
AGG_ROWS = 102400
ZROWS = AGG_ROWS // 16


def _sc_body(xaug_hbm, ei_hbm, zeros_hbm, out_hbm,
             idx_v0, idx_v1, rows_v0, rows_v1, agg_sh,
             gsem0, gsem1, ssem0, ssem1):
    c = lax.axis_index("c")
    s = lax.axis_index("s")
    w = c * 16 + s

    pltpu.sync_copy(zeros_hbm, agg_sh.at[pl.ds(s * ZROWS, ZROWS)])
    plsc.subcore_barrier()

    base_chunk = w * CHUNKS_MAIN
    n = jnp.where(w == NUM_WORKERS - 1, CHUNKS_LAST, CHUNKS_MAIN)

    def load_idx(i, idx_v):
        r0 = (base_chunk + i) * ROWS_PER_ITER
        pltpu.sync_copy(ei_hbm.at[:, pl.ds(r0, ROWS_PER_ITER)], idx_v)

    def fire_gathers(idx_v, rows_v, gsem):
        for j in range(ROWS_PER_ITER):
            pltpu.async_copy(xaug_hbm.at[idx_v.at[1].at[j]],
                             rows_v.at[j], gsem)

    def drain_gathers(idx_v, rows_v, gsem):
        for j in range(ROWS_PER_ITER):
            pltpu.make_async_copy(xaug_hbm.at[idx_v.at[1].at[j]],
                                  rows_v.at[j], gsem).wait()

    def fire_scatters(idx_v, rows_v, ssem):
        for j in range(ROWS_PER_ITER):
            pltpu.async_copy(rows_v.at[j],
                             agg_sh.at[idx_v.at[0].at[j]], ssem, add=True)

    def drain_scatters(idx_v, rows_v, ssem):
        for j in range(ROWS_PER_ITER):
            pltpu.make_async_copy(rows_v.at[j],
                                  agg_sh.at[idx_v.at[0].at[j]], ssem).wait()

    bufs = ((idx_v0, rows_v0, gsem0, ssem0),
            (idx_v1, rows_v1, gsem1, ssem1))

    load_idx(0, idx_v0)
    fire_gathers(idx_v0, rows_v0, gsem0)

    def stage(i, cur, nxt):
        idx_c, rows_c, gsem_c, ssem_c = cur
        idx_n, rows_n, gsem_n, ssem_n = nxt

        @pl.when(i >= 1)
        def _():
            drain_scatters(idx_n, rows_n, ssem_n)

        @pl.when(i + 1 < n)
        def _():
            load_idx(i + 1, idx_n)

        drain_gathers(idx_c, rows_c, gsem_c)

        @pl.when(i + 1 < n)
        def _():
            fire_gathers(idx_n, rows_n, gsem_n)

        fire_scatters(idx_c, rows_c, ssem_c)

    @pl.loop(0, n)
    def _edge_iter(i):
        even = (i % 2) == 0

        @pl.when(even)
        def _():
            stage(i, bufs[0], bufs[1])

        @pl.when(jnp.logical_not(even))
        def _():
            stage(i, bufs[1], bufs[0])

    last_even = ((n - 1) % 2) == 0

    @pl.when(last_even)
    def _():
        drain_scatters(idx_v0, rows_v0, ssem0)

    @pl.when(jnp.logical_not(last_even))
    def _():
        drain_scatters(idx_v1, rows_v1, ssem1)

    plsc.subcore_barrier()
    pltpu.sync_copy(agg_sh.at[pl.ds(s * ZROWS, ZROWS)],
                    out_hbm.at[c].at[pl.ds(s * ZROWS, ZROWS)])


def _scatter_parts(xaug, ei3, zeros_hbm):
    mesh = plsc.VectorSubcoreMesh(core_axis_name="c", subcore_axis_name="s")
    f = pl.kernel(
        _sc_body,
        out_type=jax.ShapeDtypeStruct((2, AGG_ROWS, XCOLS), jnp.float32),
        mesh=mesh,
        scratch_types=[
            pltpu.VMEM((2, ROWS_PER_ITER, ROW), jnp.int32),
            pltpu.VMEM((2, ROWS_PER_ITER, ROW), jnp.int32),
            pltpu.VMEM((ROWS_PER_ITER, ROW, XCOLS), jnp.float32),
            pltpu.VMEM((ROWS_PER_ITER, ROW, XCOLS), jnp.float32),
            pltpu.VMEM_SHARED((AGG_ROWS, XCOLS), jnp.float32),
            pltpu.SemaphoreType.DMA,
            pltpu.SemaphoreType.DMA,
            pltpu.SemaphoreType.DMA,
            pltpu.SemaphoreType.DMA,
        ],
        compiler_params=pltpu.CompilerParams(use_tc_tiling_on_sc=False),
    )
    return f(xaug, ei3, zeros_hbm)


NBLK = 50
BLK = N_NODES // NBLK


def _mlp_body(parts_ref, x_ref, w1t_ref, w1p_ref, b1_ref, w2t_ref, b2_ref,
              w3t_ref, b3_ref, wv_ref, bv_ref, out_ref, acc_ref):
    i = pl.program_id(0)
    p = parts_ref[0] + parts_ref[1]
    lane = lax.broadcasted_iota(jnp.int32, (BLK, XCOLS), 1)
    deg = jnp.sum(jnp.where(lane == IN_DIM, p, 0.0), axis=1, keepdims=True)
    deg = jnp.maximum(deg, 1.0)
    aggw = lax.dot_general(p, w1p_ref[...],
                           (((1,), (0,)), ((), ()))) / deg
    xw = lax.dot_general(x_ref[...], w1t_ref[...], (((1,), (0,)), ((), ())))
    h = jnp.maximum(xw + aggw + b1_ref[...], 0.0)
    h = jnp.maximum(lax.dot_general(h, w2t_ref[...], (((1,), (0,)), ((), ())))
                    + b2_ref[...], 0.0)
    h = jnp.maximum(lax.dot_general(h, w3t_ref[...], (((1,), (0,)), ((), ())))
                    + b3_ref[...], 0.0)
    part_sum = jnp.sum(h, axis=0, keepdims=True)

    @pl.when(i == 0)
    def _():
        acc_ref[...] = part_sum

    @pl.when(i > 0)
    def _():
        acc_ref[...] = acc_ref[...] + part_sum

    @pl.when(i == NBLK - 1)
    def _():
        m = acc_ref[...] / jnp.float32(N_NODES)
        v = jnp.sum(m * wv_ref[...], axis=1, keepdims=True) + bv_ref[...]
        out_ref[...] = jnp.tanh(v)


def _mlp(parts, x, w1t, w1p, b1, w2t, b2, w3t, b3, wv, bv):
    return pl.pallas_call(
        _mlp_body,
        grid=(NBLK,),
        in_specs=[
            pl.BlockSpec((2, BLK, XCOLS), lambda i: (0, i, 0)),
            pl.BlockSpec((BLK, IN_DIM), lambda i: (i, 0)),
            pl.BlockSpec((IN_DIM, HIDDEN), lambda i: (0, 0)),
            pl.BlockSpec((XCOLS, HIDDEN), lambda i: (0, 0)),
            pl.BlockSpec((1, HIDDEN), lambda i: (0, 0)),
            pl.BlockSpec((HIDDEN, HIDDEN), lambda i: (0, 0)),
            pl.BlockSpec((1, HIDDEN), lambda i: (0, 0)),
            pl.BlockSpec((HIDDEN, HIDDEN), lambda i: (0, 0)),
            pl.BlockSpec((1, HIDDEN), lambda i: (0, 0)),
            pl.BlockSpec((1, HIDDEN), lambda i: (0, 0)),
            pl.BlockSpec((1, 1), lambda i: (0, 0)),
        ],
        out_specs=pl.BlockSpec((1, 1), lambda i: (0, 0)),
        out_shape=jax.ShapeDtypeStruct((1, 1), jnp.float32),
        scratch_shapes=[pltpu.VMEM((1, HIDDEN), jnp.float32)],
        compiler_params=pltpu.CompilerParams(
            dimension_semantics=("arbitrary",)),
    )(parts, x, w1t, w1p, b1, w2t, b2, w3t, b3, wv, bv)


def kernel(x, edge_index, W1, b1, W2, b2, W3, b3, Wv, bv):
    ei3 = edge_index.reshape(2, EDGE_ROWS, ROW)
    xaug = jnp.zeros((N_NODES, XCOLS), dtype=jnp.float32)
    xaug = xaug.at[:, 0:IN_DIM].set(x).at[:, IN_DIM].set(1.0)
    zeros_hbm = jnp.zeros((ZROWS, XCOLS), dtype=jnp.float32)

    parts = _scatter_parts(xaug, ei3, zeros_hbm)

    w1p = jnp.zeros((XCOLS, HIDDEN), dtype=jnp.float32).at[0:IN_DIM].set(W1.T)
    out = _mlp(parts, x, W1.T, w1p, b1.reshape(1, -1), W2.T,
               b2.reshape(1, -1), W3.T, b3.reshape(1, -1), Wv.reshape(1, -1),
               bv.reshape(1, 1))
    return jnp.squeeze(out)


# speedup vs baseline: 1.3463x; 1.0892x over previous
import functools

import jax
import jax.numpy as jnp
from jax import lax
from jax.experimental import pallas as pl
from jax.experimental.pallas import tpu as pltpu
from jax.experimental.pallas import tpu_sc as plsc

N_NODES = 100000
N_EDGES = 6400000
IN_DIM = 8
HIDDEN = 128
XCOLS = 16

NUM_WORKERS = 32
ROW = 128
ROWS_PER_ITER = 4
EDGE_ROWS = N_EDGES // ROW
N_CHUNKS = EDGE_ROWS // ROWS_PER_ITER
CHUNKS_MAIN = -(-N_CHUNKS // NUM_WORKERS)
CHUNKS_LAST = N_CHUNKS - 31 * CHUNKS_MAIN
AGG_ROWS = 102400
ZROWS = AGG_ROWS // 16


def _sc_body(xaug_hbm, ei_hbm, zeros_hbm, out_hbm,
             idx_v0, idx_v1, rows_v0, rows_v1, agg_sh,
             gsem0, gsem1, ssem0, ssem1):
    c = lax.axis_index("c")
    s = lax.axis_index("s")
    w = c * 16 + s

    pltpu.sync_copy(zeros_hbm, agg_sh.at[pl.ds(s * ZROWS, ZROWS)])
    plsc.subcore_barrier()

    base_chunk = w * CHUNKS_MAIN
    n = jnp.where(w == NUM_WORKERS - 1, CHUNKS_LAST, CHUNKS_MAIN)

    def load_idx(i, idx_v):
        r0 = (base_chunk + i) * ROWS_PER_ITER
        pltpu.sync_copy(ei_hbm.at[:, pl.ds(r0, ROWS_PER_ITER)], idx_v)

    def fire_gathers(idx_v, rows_v, gsem):
        for j in range(ROWS_PER_ITER):
            pltpu.async_copy(xaug_hbm.at[idx_v.at[1].at[j]],
                             rows_v.at[j], gsem)

    def drain_gathers(idx_v, rows_v, gsem):
        for j in range(ROWS_PER_ITER):
            pltpu.make_async_copy(xaug_hbm.at[idx_v.at[1].at[j]],
                                  rows_v.at[j], gsem).wait()

    def fire_scatters(idx_v, rows_v, ssem):
        for j in range(ROWS_PER_ITER):
            pltpu.async_copy(rows_v.at[j],
                             agg_sh.at[idx_v.at[0].at[j]], ssem, add=True)

    def drain_scatters(idx_v, rows_v, ssem):
        for j in range(ROWS_PER_ITER):
            pltpu.make_async_copy(rows_v.at[j],
                                  agg_sh.at[idx_v.at[0].at[j]], ssem).wait()

    bufs = ((idx_v0, rows_v0, gsem0, ssem0),
            (idx_v1, rows_v1, gsem1, ssem1))

    load_idx(0, idx_v0)
    fire_gathers(idx_v0, rows_v0, gsem0)

    def stage(i, cur, nxt):
        idx_c, rows_c, gsem_c, ssem_c = cur
        idx_n, rows_n, gsem_n, ssem_n = nxt

        @pl.when(i >= 1)
        def _():
            drain_scatters(idx_n, rows_n, ssem_n)

        @pl.when(i + 1 < n)
        def _():
            load_idx(i + 1, idx_n)

        drain_gathers(idx_c, rows_c, gsem_c)

        @pl.when(i + 1 < n)
        def _():
            fire_gathers(idx_n, rows_n, gsem_n)

        fire_scatters(idx_c, rows_c, ssem_c)

    @pl.loop(0, n)
    def _edge_iter(i):
        even = (i % 2) == 0

        @pl.when(even)
        def _():
            stage(i, bufs[0], bufs[1])

        @pl.when(jnp.logical_not(even))
        def _():
            stage(i, bufs[1], bufs[0])

    last_even = ((n - 1) % 2) == 0

    @pl.when(last_even)
    def _():
        drain_scatters(idx_v0, rows_v0, ssem0)

    @pl.when(jnp.logical_not(last_even))
    def _():
        drain_scatters(idx_v1, rows_v1, ssem1)

    plsc.subcore_barrier()
    pltpu.sync_copy(agg_sh.at[pl.ds(s * ZROWS, ZROWS)],
                    out_hbm.at[c].at[pl.ds(s * ZROWS, ZROWS)])


def _scatter_parts(xaug, ei3, zeros_hbm):
    mesh = plsc.VectorSubcoreMesh(core_axis_name="c", subcore_axis_name="s")
    f = pl.kernel(
        _sc_body,
        out_type=jax.ShapeDtypeStruct((2, AGG_ROWS, XCOLS), jnp.float32),
        mesh=mesh,
        scratch_types=[
            pltpu.VMEM((2, ROWS_PER_ITER, ROW), jnp.int32),
            pltpu.VMEM((2, ROWS_PER_ITER, ROW), jnp.int32),
            pltpu.VMEM((ROWS_PER_ITER, ROW, XCOLS), jnp.float32),
            pltpu.VMEM((ROWS_PER_ITER, ROW, XCOLS), jnp.float32),
            pltpu.VMEM_SHARED((AGG_ROWS, XCOLS), jnp.float32),
            pltpu.SemaphoreType.DMA,
            pltpu.SemaphoreType.DMA,
            pltpu.SemaphoreType.DMA,
            pltpu.SemaphoreType.DMA,
        ],
        compiler_params=pltpu.CompilerParams(use_tc_tiling_on_sc=False),
    )
    return f(xaug, ei3, zeros_hbm)


PACK = 128 // XCOLS
PROWS = AGG_ROWS // PACK
NBLK = 50
PBLK = PROWS // NBLK
BLK = PBLK * PACK
PWIDE = PACK * HIDDEN


def _mlp_body(parts_ref, xa_ref, w1pp_ref, e8p_ref, b1p_ref, w2t_ref,
              b2_ref, w3t_ref, b3_ref, wv_ref, bv_ref, out_ref, acc_ref):
    i = pl.program_id(0)
    p2 = parts_ref[0] + parts_ref[1]
    dims = (((1,), (0,)), ((), ()))
    degp = jnp.maximum(lax.dot_general(p2, e8p_ref[...], dims), 1.0)
    aggw = lax.dot_general(p2, w1pp_ref[...], dims) / degp
    xw = lax.dot_general(xa_ref[...], w1pp_ref[...], dims)
    h1p = jnp.maximum(xw + aggw + b1p_ref[...], 0.0)
    h = h1p.reshape(BLK, HIDDEN)
    h = jnp.maximum(lax.dot_general(h, w2t_ref[...], dims)
                    + b2_ref[...], 0.0)
    h = jnp.maximum(lax.dot_general(h, w3t_ref[...], dims)
                    + b3_ref[...], 0.0)
    node = lax.broadcasted_iota(jnp.int32, (BLK, HIDDEN), 0) + i * BLK
    h = jnp.where(node < N_NODES, h, 0.0)
    part_sum = jnp.sum(h, axis=0, keepdims=True)

    @pl.when(i == 0)
    def _():
        acc_ref[...] = part_sum

    @pl.when(i > 0)
    def _():
        acc_ref[...] = acc_ref[...] + part_sum

    @pl.when(i == NBLK - 1)
    def _():
        m = acc_ref[...] / jnp.float32(N_NODES)
        v = jnp.sum(m * wv_ref[...], axis=1, keepdims=True) + bv_ref[...]
        out_ref[...] = jnp.tanh(v)


def _mlp(parts_p, xa_p, w1pp, e8p, b1p, w2t, b2, w3t, b3, wv, bv):
    return pl.pallas_call(
        _mlp_body,
        grid=(NBLK,),
        in_specs=[
            pl.BlockSpec((2, PBLK, 128), lambda i: (0, i, 0)),
            pl.BlockSpec((PBLK, 128), lambda i: (i, 0)),
            pl.BlockSpec((128, PWIDE), lambda i: (0, 0)),
            pl.BlockSpec((128, PWIDE), lambda i: (0, 0)),
            pl.BlockSpec((1, PWIDE), lambda i: (0, 0)),
            pl.BlockSpec((HIDDEN, HIDDEN), lambda i: (0, 0)),
            pl.BlockSpec((1, HIDDEN), lambda i: (0, 0)),
            pl.BlockSpec((HIDDEN, HIDDEN), lambda i: (0, 0)),
            pl.BlockSpec((1, HIDDEN), lambda i: (0, 0)),
            pl.BlockSpec((1, HIDDEN), lambda i: (0, 0)),
            pl.BlockSpec((1, 1), lambda i: (0, 0)),
        ],
        out_specs=pl.BlockSpec((1, 1), lambda i: (0, 0)),
        out_shape=jax.ShapeDtypeStruct((1, 1), jnp.float32),
        scratch_shapes=[pltpu.VMEM((1, HIDDEN), jnp.float32)],
        compiler_params=pltpu.CompilerParams(
            dimension_semantics=("arbitrary",)),
    )(parts_p, xa_p, w1pp, e8p, b1p, w2t, b2, w3t, b3, wv, bv)


def kernel(x, edge_index, W1, b1, W2, b2, W3, b3, Wv, bv):
    ei3 = edge_index.reshape(2, EDGE_ROWS, ROW)
    xaug = jnp.zeros((AGG_ROWS, XCOLS), dtype=jnp.float32)
    xaug = xaug.at[0:N_NODES, 0:IN_DIM].set(x)
    xaug = xaug.at[0:N_NODES, IN_DIM].set(1.0)
    zeros_hbm = jnp.zeros((ZROWS, XCOLS), dtype=jnp.float32)

    parts = _scatter_parts(xaug, ei3, zeros_hbm)

    parts_p = parts.reshape(2, PROWS, 128)
    xa_p = xaug.reshape(PROWS, 128)

    w1p = jnp.zeros((XCOLS, HIDDEN), dtype=jnp.float32).at[0:IN_DIM].set(W1.T)
    w1pp = jnp.zeros((128, PWIDE), dtype=jnp.float32)
    e8p = jnp.zeros((128, PWIDE), dtype=jnp.float32)
    for j in range(PACK):
        w1pp = lax.dynamic_update_slice(w1pp, w1p, (j * XCOLS, j * HIDDEN))
        e8p = lax.dynamic_update_slice(
            e8p, jnp.ones((1, HIDDEN), jnp.float32),
            (j * XCOLS + IN_DIM, j * HIDDEN))
    b1p = jnp.tile(b1.reshape(1, -1), (1, PACK))

    out = _mlp(parts_p, xa_p, w1pp, e8p, b1p, W2.T,
               b2.reshape(1, -1), W3.T, b3.reshape(1, -1), Wv.reshape(1, -1),
               bv.reshape(1, 1))
    return jnp.squeeze(out)


# speedup vs baseline: 1.6643x; 1.2362x over previous
import functools

import jax
import jax.numpy as jnp
from jax import lax
from jax.experimental import pallas as pl
from jax.experimental.pallas import tpu as pltpu
from jax.experimental.pallas import tpu_sc as plsc

N_NODES = 100000
N_EDGES = 6400000
IN_DIM = 8
HIDDEN = 128
XCOLS = 16

NUM_WORKERS = 32
ROW = 128
ROWS_PER_ITER = 4
EDGE_ROWS = N_EDGES // ROW
N_CHUNKS = EDGE_ROWS // ROWS_PER_ITER
CHUNKS_MAIN = -(-N_CHUNKS // NUM_WORKERS)
CHUNKS_LAST = N_CHUNKS - 31 * CHUNKS_MAIN
AGG_ROWS = 102400
ZROWS = AGG_ROWS // 16


def _sc_body(xaug_hbm, ei_hbm, zeros_hbm, out_hbm,
             idx_v0, idx_v1, rows_v0, rows_v1, agg_sh,
             gsem0, gsem1, ssem0, ssem1):
    c = lax.axis_index("c")
    s = lax.axis_index("s")
    w = c * 16 + s

    pltpu.sync_copy(zeros_hbm, agg_sh.at[pl.ds(s * ZROWS, ZROWS)])
    plsc.subcore_barrier()

    base_chunk = w * CHUNKS_MAIN
    n = jnp.where(w == NUM_WORKERS - 1, CHUNKS_LAST, CHUNKS_MAIN)

    def load_idx(i, idx_v):
        r0 = (base_chunk + i) * ROWS_PER_ITER
        pltpu.sync_copy(ei_hbm.at[:, pl.ds(r0, ROWS_PER_ITER)], idx_v)

    def fire_gathers(idx_v, rows_v, gsem):
        for j in range(ROWS_PER_ITER):
            pltpu.async_copy(xaug_hbm.at[idx_v.at[1].at[j]],
                             rows_v.at[j], gsem)

    def drain_gathers(idx_v, rows_v, gsem):
        for j in range(ROWS_PER_ITER):
            pltpu.make_async_copy(xaug_hbm.at[idx_v.at[1].at[j]],
                                  rows_v.at[j], gsem).wait()

    def fire_scatters(idx_v, rows_v, ssem):
        for j in range(ROWS_PER_ITER):
            pltpu.async_copy(rows_v.at[j],
                             agg_sh.at[idx_v.at[0].at[j]], ssem, add=True)

    def drain_scatters(idx_v, rows_v, ssem):
        for j in range(ROWS_PER_ITER):
            pltpu.make_async_copy(rows_v.at[j],
                                  agg_sh.at[idx_v.at[0].at[j]], ssem).wait()

    bufs = ((idx_v0, rows_v0, gsem0, ssem0),
            (idx_v1, rows_v1, gsem1, ssem1))

    load_idx(0, idx_v0)
    fire_gathers(idx_v0, rows_v0, gsem0)

    def stage(i, cur, nxt):
        idx_c, rows_c, gsem_c, ssem_c = cur
        idx_n, rows_n, gsem_n, ssem_n = nxt

        @pl.when(i >= 1)
        def _():
            drain_scatters(idx_n, rows_n, ssem_n)

        @pl.when(i + 1 < n)
        def _():
            load_idx(i + 1, idx_n)

        drain_gathers(idx_c, rows_c, gsem_c)

        @pl.when(i + 1 < n)
        def _():
            fire_gathers(idx_n, rows_n, gsem_n)

        fire_scatters(idx_c, rows_c, ssem_c)

    @pl.loop(0, n)
    def _edge_iter(i):
        even = (i % 2) == 0

        @pl.when(even)
        def _():
            stage(i, bufs[0], bufs[1])

        @pl.when(jnp.logical_not(even))
        def _():
            stage(i, bufs[1], bufs[0])

    last_even = ((n - 1) % 2) == 0

    @pl.when(last_even)
    def _():
        drain_scatters(idx_v0, rows_v0, ssem0)

    @pl.when(jnp.logical_not(last_even))
    def _():
        drain_scatters(idx_v1, rows_v1, ssem1)

    plsc.subcore_barrier()
    pltpu.sync_copy(agg_sh.at[pl.ds(s * ZROWS, ZROWS)],
                    out_hbm.at[c].at[pl.ds(s * ZROWS, ZROWS)])


def _scatter_parts(xaug, ei3, zeros_hbm):
    mesh = plsc.VectorSubcoreMesh(core_axis_name="c", subcore_axis_name="s")
    f = pl.kernel(
        _sc_body,
        out_type=jax.ShapeDtypeStruct((2, AGG_ROWS, XCOLS), jnp.float32),
        mesh=mesh,
        scratch_types=[
            pltpu.VMEM((2, ROWS_PER_ITER, ROW), jnp.int32),
            pltpu.VMEM((2, ROWS_PER_ITER, ROW), jnp.int32),
            pltpu.VMEM((ROWS_PER_ITER, ROW, XCOLS), jnp.float32),
            pltpu.VMEM((ROWS_PER_ITER, ROW, XCOLS), jnp.float32),
            pltpu.VMEM_SHARED((AGG_ROWS, XCOLS), jnp.float32),
            pltpu.SemaphoreType.DMA,
            pltpu.SemaphoreType.DMA,
            pltpu.SemaphoreType.DMA,
            pltpu.SemaphoreType.DMA,
        ],
        compiler_params=pltpu.CompilerParams(use_tc_tiling_on_sc=False),
    )
    return f(xaug, ei3, zeros_hbm)


PACK = 128 // XCOLS
PROWS = AGG_ROWS // PACK
NBLK = 50
PBLK = PROWS // NBLK
BLK = PBLK * PACK
PWIDE = PACK * HIDDEN


def _mlp_body(parts_ref, xa_ref, w1pp_ref, e8p_ref, b1p_ref, w2t_ref,
              b2_ref, w3t_ref, b3_ref, wv_ref, bv_ref, out_ref, acc_ref):
    i = pl.program_id(0)
    p2 = parts_ref[0] + parts_ref[1]
    dims = (((1,), (0,)), ((), ()))
    degp = jnp.maximum(lax.dot_general(p2, e8p_ref[...], dims), 1.0)
    aggw = lax.dot_general(p2, w1pp_ref[...], dims) / degp
    xw = lax.dot_general(xa_ref[...], w1pp_ref[...], dims)
    h1p = jnp.maximum(xw + aggw + b1p_ref[...], 0.0)
    h = h1p.reshape(BLK, HIDDEN)
    h = jnp.maximum(lax.dot_general(h, w2t_ref[...], dims)
                    + b2_ref[...], 0.0)
    h = jnp.maximum(lax.dot_general(h, w3t_ref[...], dims)
                    + b3_ref[...], 0.0)
    node = lax.broadcasted_iota(jnp.int32, (BLK, HIDDEN), 0) + i * BLK
    h = jnp.where(node < N_NODES, h, 0.0)
    part_sum = jnp.sum(h, axis=0, keepdims=True)

    @pl.when(i == 0)
    def _():
        acc_ref[...] = part_sum

    @pl.when(i > 0)
    def _():
        acc_ref[...] = acc_ref[...] + part_sum

    @pl.when(i == NBLK - 1)
    def _():
        m = acc_ref[...] / jnp.float32(N_NODES)
        v = jnp.sum(m * wv_ref[...], axis=1, keepdims=True) + bv_ref[...]
        out_ref[...] = jnp.tanh(v)


def _mlp(parts_p, xa_p, w1pp, e8p, b1p, w2t, b2, w3t, b3, wv, bv):
    return pl.pallas_call(
        _mlp_body,
        grid=(NBLK,),
        in_specs=[
            pl.BlockSpec((2, PBLK, 128), lambda i: (0, i, 0)),
            pl.BlockSpec((PBLK, 128), lambda i: (i, 0)),
            pl.BlockSpec((128, PWIDE), lambda i: (0, 0)),
            pl.BlockSpec((128, PWIDE), lambda i: (0, 0)),
            pl.BlockSpec((1, PWIDE), lambda i: (0, 0)),
            pl.BlockSpec((HIDDEN, HIDDEN), lambda i: (0, 0)),
            pl.BlockSpec((1, HIDDEN), lambda i: (0, 0)),
            pl.BlockSpec((HIDDEN, HIDDEN), lambda i: (0, 0)),
            pl.BlockSpec((1, HIDDEN), lambda i: (0, 0)),
            pl.BlockSpec((1, HIDDEN), lambda i: (0, 0)),
            pl.BlockSpec((1, 1), lambda i: (0, 0)),
        ],
        out_specs=pl.BlockSpec((1, 1), lambda i: (0, 0)),
        out_shape=jax.ShapeDtypeStruct((1, 1), jnp.float32),
        scratch_shapes=[pltpu.VMEM((1, HIDDEN), jnp.float32)],
        compiler_params=pltpu.CompilerParams(
            dimension_semantics=("arbitrary",)),
    )(parts_p, xa_p, w1pp, e8p, b1p, w2t, b2, w3t, b3, wv, bv)


def kernel(x, edge_index, W1, b1, W2, b2, W3, b3, Wv, bv):
    ei3 = edge_index.reshape(2, EDGE_ROWS, ROW)
    x3 = x.reshape(N_NODES // PACK, PACK, IN_DIM)
    x3 = jnp.concatenate(
        [x3,
         jnp.ones((N_NODES // PACK, PACK, 1), jnp.float32),
         jnp.zeros((N_NODES // PACK, PACK, XCOLS - IN_DIM - 1), jnp.float32)],
        axis=2)
    x3 = jnp.pad(x3, ((0, PROWS - N_NODES // PACK), (0, 0), (0, 0)))
    xa_p = x3.reshape(PROWS, 128)
    xaug = xa_p.reshape(AGG_ROWS, XCOLS)
    zeros_hbm = jnp.zeros((ZROWS, XCOLS), dtype=jnp.float32)

    parts = _scatter_parts(xaug, ei3, zeros_hbm)
    parts_p = parts.reshape(2, PROWS, 128)

    w1p = jnp.zeros((XCOLS, HIDDEN), dtype=jnp.float32).at[0:IN_DIM].set(W1.T)
    w1pp = jnp.zeros((128, PWIDE), dtype=jnp.float32)
    e8p = jnp.zeros((128, PWIDE), dtype=jnp.float32)
    for j in range(PACK):
        w1pp = lax.dynamic_update_slice(w1pp, w1p, (j * XCOLS, j * HIDDEN))
        e8p = lax.dynamic_update_slice(
            e8p, jnp.ones((1, HIDDEN), jnp.float32),
            (j * XCOLS + IN_DIM, j * HIDDEN))
    b1p = jnp.tile(b1.reshape(1, -1), (1, PACK))

    out = _mlp(parts_p, xa_p, w1pp, e8p, b1p, W2.T,
               b2.reshape(1, -1), W3.T, b3.reshape(1, -1), Wv.reshape(1, -1),
               bv.reshape(1, 1))
    return jnp.squeeze(out)


# speedup vs baseline: 1.6865x; 1.0133x over previous
import functools

import jax
import jax.numpy as jnp
from jax import lax
from jax.experimental import pallas as pl
from jax.experimental.pallas import tpu as pltpu
from jax.experimental.pallas import tpu_sc as plsc

N_NODES = 100000
N_EDGES = 6400000
IN_DIM = 8
HIDDEN = 128
XCOLS = 16

NUM_WORKERS = 32
ROW = 128
ROWS_PER_ITER = 4
EDGE_ROWS = N_EDGES // ROW
N_CHUNKS = EDGE_ROWS // ROWS_PER_ITER
CHUNKS_MAIN = -(-N_CHUNKS // NUM_WORKERS)
CHUNKS_LAST = N_CHUNKS - 31 * CHUNKS_MAIN
AGG_ROWS = 102400
ZROWS = AGG_ROWS // 16


def _sc_body(xaug_hbm, ei_hbm, zeros_hbm, out_hbm,
             idx_v0, idx_v1, rows_v0, rows_v1, agg_sh,
             gsem0, gsem1, ssem0, ssem1):
    c = lax.axis_index("c")
    s = lax.axis_index("s")
    w = c * 16 + s

    pltpu.sync_copy(zeros_hbm, agg_sh.at[pl.ds(s * ZROWS, ZROWS)])
    plsc.subcore_barrier()

    base_chunk = w * CHUNKS_MAIN
    n = jnp.where(w == NUM_WORKERS - 1, CHUNKS_LAST, CHUNKS_MAIN)

    def load_idx(i, idx_v):
        r0 = (base_chunk + i) * ROWS_PER_ITER
        pltpu.sync_copy(ei_hbm.at[:, pl.ds(r0, ROWS_PER_ITER)], idx_v)

    def fire_gathers(idx_v, rows_v, gsem):
        for j in range(ROWS_PER_ITER):
            pltpu.async_copy(xaug_hbm.at[idx_v.at[1].at[j]],
                             rows_v.at[j], gsem)

    def drain_gathers(idx_v, rows_v, gsem):
        for j in range(ROWS_PER_ITER):
            pltpu.make_async_copy(xaug_hbm.at[idx_v.at[1].at[j]],
                                  rows_v.at[j], gsem).wait()

    def fire_scatters(idx_v, rows_v, ssem):
        for j in range(ROWS_PER_ITER):
            pltpu.async_copy(rows_v.at[j],
                             agg_sh.at[idx_v.at[0].at[j]], ssem, add=True)

    def drain_scatters(idx_v, rows_v, ssem):
        for j in range(ROWS_PER_ITER):
            pltpu.make_async_copy(rows_v.at[j],
                                  agg_sh.at[idx_v.at[0].at[j]], ssem).wait()

    bufs = ((idx_v0, rows_v0, gsem0, ssem0),
            (idx_v1, rows_v1, gsem1, ssem1))

    load_idx(0, idx_v0)
    fire_gathers(idx_v0, rows_v0, gsem0)

    def stage(i, cur, nxt):
        idx_c, rows_c, gsem_c, ssem_c = cur
        idx_n, rows_n, gsem_n, ssem_n = nxt

        @pl.when(i >= 1)
        def _():
            drain_scatters(idx_n, rows_n, ssem_n)

        @pl.when(i + 1 < n)
        def _():
            load_idx(i + 1, idx_n)

        drain_gathers(idx_c, rows_c, gsem_c)

        @pl.when(i + 1 < n)
        def _():
            fire_gathers(idx_n, rows_n, gsem_n)

        fire_scatters(idx_c, rows_c, ssem_c)

    @pl.loop(0, n)
    def _edge_iter(i):
        even = (i % 2) == 0

        @pl.when(even)
        def _():
            stage(i, bufs[0], bufs[1])

        @pl.when(jnp.logical_not(even))
        def _():
            stage(i, bufs[1], bufs[0])

    last_even = ((n - 1) % 2) == 0

    @pl.when(last_even)
    def _():
        drain_scatters(idx_v0, rows_v0, ssem0)

    @pl.when(jnp.logical_not(last_even))
    def _():
        drain_scatters(idx_v1, rows_v1, ssem1)

    plsc.subcore_barrier()
    pltpu.sync_copy(agg_sh.at[pl.ds(s * ZROWS, ZROWS)],
                    out_hbm.at[c].at[pl.ds(s * ZROWS, ZROWS)])


def _scatter_parts(xaug, ei3, zeros_hbm):
    mesh = plsc.VectorSubcoreMesh(core_axis_name="c", subcore_axis_name="s")
    f = pl.kernel(
        _sc_body,
        out_type=jax.ShapeDtypeStruct((2, AGG_ROWS, XCOLS), jnp.float32),
        mesh=mesh,
        scratch_types=[
            pltpu.VMEM((2, ROWS_PER_ITER, ROW), jnp.int32),
            pltpu.VMEM((2, ROWS_PER_ITER, ROW), jnp.int32),
            pltpu.VMEM((ROWS_PER_ITER, ROW, XCOLS), jnp.float32),
            pltpu.VMEM((ROWS_PER_ITER, ROW, XCOLS), jnp.float32),
            pltpu.VMEM_SHARED((AGG_ROWS, XCOLS), jnp.float32),
            pltpu.SemaphoreType.DMA,
            pltpu.SemaphoreType.DMA,
            pltpu.SemaphoreType.DMA,
            pltpu.SemaphoreType.DMA,
        ],
        compiler_params=pltpu.CompilerParams(use_tc_tiling_on_sc=False),
    )
    return f(xaug, ei3, zeros_hbm)


PACK = 128 // XCOLS
PROWS = AGG_ROWS // PACK
NBLK = 50
PBLK = PROWS // NBLK
BLK = PBLK * PACK
PWIDE = PACK * HIDDEN


def _mlp_body(parts_ref, xa_ref, w1pp_ref, e8s_ref, b1p_ref, w2t_ref,
              b2_ref, w3t_ref, b3_ref, wv_ref, bv_ref, out_ref, acc_ref):
    i = pl.program_id(0)
    p2 = parts_ref[0] + parts_ref[1]
    dims = (((1,), (0,)), ((), ()))
    degp = jnp.maximum(lax.dot_general(p2, e8s_ref[...], dims), 1.0)
    q = xa_ref[...] + p2 / degp
    h1p = jnp.maximum(lax.dot_general(q, w1pp_ref[...], dims)
                      + b1p_ref[...], 0.0)
    h = h1p.reshape(BLK, HIDDEN)
    h = jnp.maximum(lax.dot_general(h, w2t_ref[...], dims)
                    + b2_ref[...], 0.0)
    h = jnp.maximum(lax.dot_general(h, w3t_ref[...], dims)
                    + b3_ref[...], 0.0)
    node = lax.broadcasted_iota(jnp.int32, (BLK, HIDDEN), 0) + i * BLK
    h = jnp.where(node < N_NODES, h, 0.0)
    part_sum = jnp.sum(h, axis=0, keepdims=True)

    @pl.when(i == 0)
    def _():
        acc_ref[...] = part_sum

    @pl.when(i > 0)
    def _():
        acc_ref[...] = acc_ref[...] + part_sum

    @pl.when(i == NBLK - 1)
    def _():
        m = acc_ref[...] / jnp.float32(N_NODES)
        v = jnp.sum(m * wv_ref[...], axis=1, keepdims=True) + bv_ref[...]
        out_ref[...] = jnp.tanh(v)


def _mlp(parts_p, xa_p, w1pp, e8s, b1p, w2t, b2, w3t, b3, wv, bv):
    return pl.pallas_call(
        _mlp_body,
        grid=(NBLK,),
        in_specs=[
            pl.BlockSpec((2, PBLK, 128), lambda i: (0, i, 0)),
            pl.BlockSpec((PBLK, 128), lambda i: (i, 0)),
            pl.BlockSpec((128, PWIDE), lambda i: (0, 0)),
            pl.BlockSpec((128, 128), lambda i: (0, 0)),
            pl.BlockSpec((1, PWIDE), lambda i: (0, 0)),
            pl.BlockSpec((HIDDEN, HIDDEN), lambda i: (0, 0)),
            pl.BlockSpec((1, HIDDEN), lambda i: (0, 0)),
            pl.BlockSpec((HIDDEN, HIDDEN), lambda i: (0, 0)),
            pl.BlockSpec((1, HIDDEN), lambda i: (0, 0)),
            pl.BlockSpec((1, HIDDEN), lambda i: (0, 0)),
            pl.BlockSpec((1, 1), lambda i: (0, 0)),
        ],
        out_specs=pl.BlockSpec((1, 1), lambda i: (0, 0)),
        out_shape=jax.ShapeDtypeStruct((1, 1), jnp.float32),
        scratch_shapes=[pltpu.VMEM((1, HIDDEN), jnp.float32)],
        compiler_params=pltpu.CompilerParams(
            dimension_semantics=("arbitrary",)),
    )(parts_p, xa_p, w1pp, e8s, b1p, w2t, b2, w3t, b3, wv, bv)


def kernel(x, edge_index, W1, b1, W2, b2, W3, b3, Wv, bv):
    ei3 = edge_index.reshape(2, EDGE_ROWS, ROW)
    x3 = x.reshape(N_NODES // PACK, PACK, IN_DIM)
    x3 = jnp.concatenate(
        [x3,
         jnp.ones((N_NODES // PACK, PACK, 1), jnp.float32),
         jnp.zeros((N_NODES // PACK, PACK, XCOLS - IN_DIM - 1), jnp.float32)],
        axis=2)
    x3 = jnp.pad(x3, ((0, PROWS - N_NODES // PACK), (0, 0), (0, 0)))
    xa_p = x3.reshape(PROWS, 128)
    xaug = xa_p.reshape(AGG_ROWS, XCOLS)
    zeros_hbm = jnp.zeros((ZROWS, XCOLS), dtype=jnp.float32)

    parts = _scatter_parts(xaug, ei3, zeros_hbm)
    parts_p = parts.reshape(2, PROWS, 128)

    w1p = jnp.zeros((XCOLS, HIDDEN), dtype=jnp.float32).at[0:IN_DIM].set(W1.T)
    w1pp = jnp.zeros((128, PWIDE), dtype=jnp.float32)
    e8s = jnp.zeros((128, 128), dtype=jnp.float32)
    for j in range(PACK):
        w1pp = lax.dynamic_update_slice(w1pp, w1p, (j * XCOLS, j * HIDDEN))
        e8s = lax.dynamic_update_slice(
            e8s, jnp.ones((1, XCOLS), jnp.float32),
            (j * XCOLS + IN_DIM, j * XCOLS))
    b1p = jnp.tile(b1.reshape(1, -1), (1, PACK))

    out = _mlp(parts_p, xa_p, w1pp, e8s, b1p, W2.T,
               b2.reshape(1, -1), W3.T, b3.reshape(1, -1), Wv.reshape(1, -1),
               bv.reshape(1, 1))
    return jnp.squeeze(out)


# speedup vs baseline: 1.8028x; 1.0689x over previous
import functools

import jax
import jax.numpy as jnp
from jax import lax
from jax.experimental import pallas as pl
from jax.experimental.pallas import tpu as pltpu
from jax.experimental.pallas import tpu_sc as plsc

N_NODES = 100000
N_EDGES = 6400000
IN_DIM = 8
HIDDEN = 128
XCOLS = 16

NUM_WORKERS = 32
ROW = 128
ROWS_PER_ITER = 5
EDGE_ROWS = N_EDGES // ROW
N_CHUNKS = EDGE_ROWS // ROWS_PER_ITER
CHUNKS_MAIN = -(-N_CHUNKS // NUM_WORKERS)
CHUNKS_LAST = N_CHUNKS - 31 * CHUNKS_MAIN
AGG_ROWS = 102400
ZROWS = AGG_ROWS // 16


def _sc_body(xaug_hbm, ei_hbm, zeros_hbm, out_hbm,
             idx_v0, idx_v1, rows_v0, rows_v1, agg_sh,
             gsem0, gsem1, ssem0, ssem1, isem0, isem1):
    c = lax.axis_index("c")
    s = lax.axis_index("s")
    w = c * 16 + s

    pltpu.sync_copy(zeros_hbm, agg_sh.at[pl.ds(s * ZROWS, ZROWS)])
    plsc.subcore_barrier()

    base_chunk = w * CHUNKS_MAIN
    n = jnp.where(w == NUM_WORKERS - 1, CHUNKS_LAST, CHUNKS_MAIN)

    def fire_idx(i, idx_v, isem):
        r0 = (base_chunk + i) * ROWS_PER_ITER
        pltpu.async_copy(ei_hbm.at[:, pl.ds(r0, ROWS_PER_ITER)], idx_v, isem)

    def wait_idx(i, idx_v, isem):
        r0 = (base_chunk + i) * ROWS_PER_ITER
        pltpu.make_async_copy(ei_hbm.at[:, pl.ds(r0, ROWS_PER_ITER)],
                              idx_v, isem).wait()

    def fire_gathers(idx_v, rows_v, gsem):
        for j in range(ROWS_PER_ITER):
            pltpu.async_copy(xaug_hbm.at[idx_v.at[1].at[j]],
                             rows_v.at[j], gsem)

    def drain_gathers(idx_v, rows_v, gsem):
        for j in range(ROWS_PER_ITER):
            pltpu.make_async_copy(xaug_hbm.at[idx_v.at[1].at[j]],
                                  rows_v.at[j], gsem).wait()

    def fire_scatters(idx_v, rows_v, ssem):
        for j in range(ROWS_PER_ITER):
            pltpu.async_copy(rows_v.at[j],
                             agg_sh.at[idx_v.at[0].at[j]], ssem, add=True)

    def drain_scatters(idx_v, rows_v, ssem):
        for j in range(ROWS_PER_ITER):
            pltpu.make_async_copy(rows_v.at[j],
                                  agg_sh.at[idx_v.at[0].at[j]], ssem).wait()

    bufs = ((idx_v0, rows_v0, gsem0, ssem0, isem0),
            (idx_v1, rows_v1, gsem1, ssem1, isem1))

    fire_idx(0, idx_v0, isem0)
    wait_idx(0, idx_v0, isem0)
    fire_gathers(idx_v0, rows_v0, gsem0)

    def stage(i, cur, nxt):
        idx_c, rows_c, gsem_c, ssem_c, isem_c = cur
        idx_n, rows_n, gsem_n, ssem_n, isem_n = nxt

        @pl.when(i >= 1)
        def _():
            drain_scatters(idx_n, rows_n, ssem_n)

        @pl.when(i + 1 < n)
        def _():
            fire_idx(i + 1, idx_n, isem_n)

        drain_gathers(idx_c, rows_c, gsem_c)

        @pl.when(i + 1 < n)
        def _():
            wait_idx(i + 1, idx_n, isem_n)
            fire_gathers(idx_n, rows_n, gsem_n)

        fire_scatters(idx_c, rows_c, ssem_c)

    @pl.loop(0, n)
    def _edge_iter(i):
        even = (i % 2) == 0

        @pl.when(even)
        def _():
            stage(i, bufs[0], bufs[1])

        @pl.when(jnp.logical_not(even))
        def _():
            stage(i, bufs[1], bufs[0])

    last_even = ((n - 1) % 2) == 0

    @pl.when(last_even)
    def _():
        drain_scatters(idx_v0, rows_v0, ssem0)

    @pl.when(jnp.logical_not(last_even))
    def _():
        drain_scatters(idx_v1, rows_v1, ssem1)

    plsc.subcore_barrier()
    pltpu.sync_copy(agg_sh.at[pl.ds(s * ZROWS, ZROWS)],
                    out_hbm.at[c].at[pl.ds(s * ZROWS, ZROWS)])


def _scatter_parts(xaug, ei3, zeros_hbm):
    mesh = plsc.VectorSubcoreMesh(core_axis_name="c", subcore_axis_name="s")
    f = pl.kernel(
        _sc_body,
        out_type=jax.ShapeDtypeStruct((2, AGG_ROWS, XCOLS), jnp.float32),
        mesh=mesh,
        scratch_types=[
            pltpu.VMEM((2, ROWS_PER_ITER, ROW), jnp.int32),
            pltpu.VMEM((2, ROWS_PER_ITER, ROW), jnp.int32),
            pltpu.VMEM((ROWS_PER_ITER, ROW, XCOLS), jnp.float32),
            pltpu.VMEM((ROWS_PER_ITER, ROW, XCOLS), jnp.float32),
            pltpu.VMEM_SHARED((AGG_ROWS, XCOLS), jnp.float32),
            pltpu.SemaphoreType.DMA,
            pltpu.SemaphoreType.DMA,
            pltpu.SemaphoreType.DMA,
            pltpu.SemaphoreType.DMA,
            pltpu.SemaphoreType.DMA,
            pltpu.SemaphoreType.DMA,
        ],
        compiler_params=pltpu.CompilerParams(use_tc_tiling_on_sc=False),
    )
    return f(xaug, ei3, zeros_hbm)


PACK = 128 // XCOLS
PROWS = AGG_ROWS // PACK
NBLK = 50
PBLK = PROWS // NBLK
BLK = PBLK * PACK
PWIDE = PACK * HIDDEN


def _mlp_body(parts_ref, xa_ref, w1pp_ref, e8s_ref, b1p_ref, w2t_ref,
              b2_ref, w3t_ref, b3_ref, wv_ref, bv_ref, out_ref, acc_ref):
    i = pl.program_id(0)
    p2 = parts_ref[0] + parts_ref[1]
    dims = (((1,), (0,)), ((), ()))
    degp = jnp.maximum(lax.dot_general(p2, e8s_ref[...], dims), 1.0)
    q = xa_ref[...] + p2 / degp
    h1p = jnp.maximum(lax.dot_general(q, w1pp_ref[...], dims)
                      + b1p_ref[...], 0.0)
    h = h1p.reshape(BLK, HIDDEN)
    h = jnp.maximum(lax.dot_general(h, w2t_ref[...], dims)
                    + b2_ref[...], 0.0)
    h = jnp.maximum(lax.dot_general(h, w3t_ref[...], dims)
                    + b3_ref[...], 0.0)
    node = lax.broadcasted_iota(jnp.int32, (BLK, HIDDEN), 0) + i * BLK
    h = jnp.where(node < N_NODES, h, 0.0)
    part_sum = jnp.sum(h, axis=0, keepdims=True)

    @pl.when(i == 0)
    def _():
        acc_ref[...] = part_sum

    @pl.when(i > 0)
    def _():
        acc_ref[...] = acc_ref[...] + part_sum

    @pl.when(i == NBLK - 1)
    def _():
        m = acc_ref[...] / jnp.float32(N_NODES)
        v = jnp.sum(m * wv_ref[...], axis=1, keepdims=True) + bv_ref[...]
        out_ref[...] = jnp.tanh(v)


def _mlp(parts_p, xa_p, w1pp, e8s, b1p, w2t, b2, w3t, b3, wv, bv):
    return pl.pallas_call(
        _mlp_body,
        grid=(NBLK,),
        in_specs=[
            pl.BlockSpec((2, PBLK, 128), lambda i: (0, i, 0)),
            pl.BlockSpec((PBLK, 128), lambda i: (i, 0)),
            pl.BlockSpec((128, PWIDE), lambda i: (0, 0)),
            pl.BlockSpec((128, 128), lambda i: (0, 0)),
            pl.BlockSpec((1, PWIDE), lambda i: (0, 0)),
            pl.BlockSpec((HIDDEN, HIDDEN), lambda i: (0, 0)),
            pl.BlockSpec((1, HIDDEN), lambda i: (0, 0)),
            pl.BlockSpec((HIDDEN, HIDDEN), lambda i: (0, 0)),
            pl.BlockSpec((1, HIDDEN), lambda i: (0, 0)),
            pl.BlockSpec((1, HIDDEN), lambda i: (0, 0)),
            pl.BlockSpec((1, 1), lambda i: (0, 0)),
        ],
        out_specs=pl.BlockSpec((1, 1), lambda i: (0, 0)),
        out_shape=jax.ShapeDtypeStruct((1, 1), jnp.float32),
        scratch_shapes=[pltpu.VMEM((1, HIDDEN), jnp.float32)],
        compiler_params=pltpu.CompilerParams(
            dimension_semantics=("arbitrary",)),
    )(parts_p, xa_p, w1pp, e8s, b1p, w2t, b2, w3t, b3, wv, bv)


def kernel(x, edge_index, W1, b1, W2, b2, W3, b3, Wv, bv):
    ei3 = edge_index.reshape(2, EDGE_ROWS, ROW)
    x3 = x.reshape(N_NODES // PACK, PACK, IN_DIM)
    x3 = jnp.concatenate(
        [x3,
         jnp.ones((N_NODES // PACK, PACK, 1), jnp.float32),
         jnp.zeros((N_NODES // PACK, PACK, XCOLS - IN_DIM - 1), jnp.float32)],
        axis=2)
    x3 = jnp.pad(x3, ((0, PROWS - N_NODES // PACK), (0, 0), (0, 0)))
    xa_p = x3.reshape(PROWS, 128)
    xaug = xa_p.reshape(AGG_ROWS, XCOLS)
    zeros_hbm = jnp.zeros((ZROWS, XCOLS), dtype=jnp.float32)

    parts = _scatter_parts(xaug, ei3, zeros_hbm)
    parts_p = parts.reshape(2, PROWS, 128)

    w1p = jnp.zeros((XCOLS, HIDDEN), dtype=jnp.float32).at[0:IN_DIM].set(W1.T)
    w1pp = jnp.zeros((128, PWIDE), dtype=jnp.float32)
    e8s = jnp.zeros((128, 128), dtype=jnp.float32)
    for j in range(PACK):
        w1pp = lax.dynamic_update_slice(w1pp, w1p, (j * XCOLS, j * HIDDEN))
        e8s = lax.dynamic_update_slice(
            e8s, jnp.ones((1, XCOLS), jnp.float32),
            (j * XCOLS + IN_DIM, j * XCOLS))
    b1p = jnp.tile(b1.reshape(1, -1), (1, PACK))

    out = _mlp(parts_p, xa_p, w1pp, e8s, b1p, W2.T,
               b2.reshape(1, -1), W3.T, b3.reshape(1, -1), Wv.reshape(1, -1),
               bv.reshape(1, 1))
    return jnp.squeeze(out)


# speedup vs baseline: 1.8049x; 1.0012x over previous
import functools

import jax
import jax.numpy as jnp
from jax import lax
from jax.experimental import pallas as pl
from jax.experimental.pallas import tpu as pltpu
from jax.experimental.pallas import tpu_sc as plsc

N_NODES = 100000
N_EDGES = 6400000
IN_DIM = 8
HIDDEN = 128
XCOLS = 16

NUM_WORKERS = 32
ROW = 128
ROWS_PER_ITER = 5
EDGE_ROWS = N_EDGES // ROW
N_CHUNKS = EDGE_ROWS // ROWS_PER_ITER
CHUNKS_MAIN = -(-N_CHUNKS // NUM_WORKERS)
CHUNKS_LAST = N_CHUNKS - 31 * CHUNKS_MAIN
AGG_ROWS = 102400
ZROWS = AGG_ROWS // 16


def _sc_body(xaug_hbm, ei_hbm, zeros_hbm, out_hbm,
             idx_v0, idx_v1, rows_v0, rows_v1, agg_sh,
             gsem0, gsem1, ssem0, ssem1, isem0, isem1):
    c = lax.axis_index("c")
    s = lax.axis_index("s")
    w = c * 16 + s

    pltpu.sync_copy(zeros_hbm, agg_sh.at[pl.ds(s * ZROWS, ZROWS)])
    plsc.subcore_barrier()

    base_chunk = w * CHUNKS_MAIN
    n = jnp.where(w == NUM_WORKERS - 1, CHUNKS_LAST, CHUNKS_MAIN)

    CHUNK_E = ROWS_PER_ITER * ROW

    def fire_idx(i, idx_v, isem):
        e0 = (base_chunk + i) * CHUNK_E
        pltpu.async_copy(ei_hbm.at[:, pl.ds(e0, CHUNK_E)], idx_v, isem)

    def wait_idx(i, idx_v, isem):
        e0 = (base_chunk + i) * CHUNK_E
        pltpu.make_async_copy(ei_hbm.at[:, pl.ds(e0, CHUNK_E)],
                              idx_v, isem).wait()

    def fire_gathers(idx_v, rows_v, gsem):
        for j in range(ROWS_PER_ITER):
            pltpu.async_copy(xaug_hbm.at[idx_v.at[1].at[pl.ds(j * ROW, ROW)]],
                             rows_v.at[j], gsem)

    def drain_gathers(idx_v, rows_v, gsem):
        for j in range(ROWS_PER_ITER):
            pltpu.make_async_copy(
                xaug_hbm.at[idx_v.at[1].at[pl.ds(j * ROW, ROW)]],
                rows_v.at[j], gsem).wait()

    def fire_scatters(idx_v, rows_v, ssem):
        for j in range(ROWS_PER_ITER):
            pltpu.async_copy(rows_v.at[j],
                             agg_sh.at[idx_v.at[0].at[pl.ds(j * ROW, ROW)]],
                             ssem, add=True)

    def drain_scatters(idx_v, rows_v, ssem):
        for j in range(ROWS_PER_ITER):
            pltpu.make_async_copy(
                rows_v.at[j],
                agg_sh.at[idx_v.at[0].at[pl.ds(j * ROW, ROW)]], ssem).wait()

    bufs = ((idx_v0, rows_v0, gsem0, ssem0, isem0),
            (idx_v1, rows_v1, gsem1, ssem1, isem1))

    fire_idx(0, idx_v0, isem0)
    wait_idx(0, idx_v0, isem0)
    fire_gathers(idx_v0, rows_v0, gsem0)

    def stage(i, cur, nxt):
        idx_c, rows_c, gsem_c, ssem_c, isem_c = cur
        idx_n, rows_n, gsem_n, ssem_n, isem_n = nxt

        @pl.when(i >= 1)
        def _():
            drain_scatters(idx_n, rows_n, ssem_n)

        @pl.when(i + 1 < n)
        def _():
            fire_idx(i + 1, idx_n, isem_n)

        drain_gathers(idx_c, rows_c, gsem_c)

        @pl.when(i + 1 < n)
        def _():
            wait_idx(i + 1, idx_n, isem_n)
            fire_gathers(idx_n, rows_n, gsem_n)

        fire_scatters(idx_c, rows_c, ssem_c)

    @pl.loop(0, n)
    def _edge_iter(i):
        even = (i % 2) == 0

        @pl.when(even)
        def _():
            stage(i, bufs[0], bufs[1])

        @pl.when(jnp.logical_not(even))
        def _():
            stage(i, bufs[1], bufs[0])

    last_even = ((n - 1) % 2) == 0

    @pl.when(last_even)
    def _():
        drain_scatters(idx_v0, rows_v0, ssem0)

    @pl.when(jnp.logical_not(last_even))
    def _():
        drain_scatters(idx_v1, rows_v1, ssem1)

    plsc.subcore_barrier()
    pltpu.sync_copy(agg_sh.at[pl.ds(s * ZROWS, ZROWS)],
                    out_hbm.at[c].at[pl.ds(s * ZROWS, ZROWS)])


def _scatter_parts(xaug, ei3, zeros_hbm):
    mesh = plsc.VectorSubcoreMesh(core_axis_name="c", subcore_axis_name="s")
    f = pl.kernel(
        _sc_body,
        out_type=jax.ShapeDtypeStruct((2, AGG_ROWS, XCOLS), jnp.float32),
        mesh=mesh,
        scratch_types=[
            pltpu.VMEM((2, ROWS_PER_ITER * ROW), jnp.int32),
            pltpu.VMEM((2, ROWS_PER_ITER * ROW), jnp.int32),
            pltpu.VMEM((ROWS_PER_ITER, ROW, XCOLS), jnp.float32),
            pltpu.VMEM((ROWS_PER_ITER, ROW, XCOLS), jnp.float32),
            pltpu.VMEM_SHARED((AGG_ROWS, XCOLS), jnp.float32),
            pltpu.SemaphoreType.DMA,
            pltpu.SemaphoreType.DMA,
            pltpu.SemaphoreType.DMA,
            pltpu.SemaphoreType.DMA,
            pltpu.SemaphoreType.DMA,
            pltpu.SemaphoreType.DMA,
        ],
        compiler_params=pltpu.CompilerParams(use_tc_tiling_on_sc=False),
    )
    return f(xaug, ei3, zeros_hbm)


PACK = 128 // XCOLS
PROWS = AGG_ROWS // PACK
NBLK = 50
PBLK = PROWS // NBLK
BLK = PBLK * PACK
PWIDE = PACK * HIDDEN


def _mlp_body(parts_ref, xa_ref, w1pp_ref, e8s_ref, b1p_ref, w2t_ref,
              b2_ref, w3t_ref, b3_ref, wv_ref, bv_ref, out_ref, acc_ref):
    i = pl.program_id(0)
    p2 = parts_ref[0] + parts_ref[1]
    dims = (((1,), (0,)), ((), ()))
    degp = jnp.maximum(lax.dot_general(p2, e8s_ref[...], dims), 1.0)
    q = xa_ref[...] + p2 / degp
    h1p = jnp.maximum(lax.dot_general(q, w1pp_ref[...], dims)
                      + b1p_ref[...], 0.0)
    h = h1p.reshape(BLK, HIDDEN)
    h = jnp.maximum(lax.dot_general(h, w2t_ref[...], dims)
                    + b2_ref[...], 0.0)
    h = jnp.maximum(lax.dot_general(h, w3t_ref[...], dims)
                    + b3_ref[...], 0.0)
    node = lax.broadcasted_iota(jnp.int32, (BLK, HIDDEN), 0) + i * BLK
    h = jnp.where(node < N_NODES, h, 0.0)
    part_sum = jnp.sum(h, axis=0, keepdims=True)

    @pl.when(i == 0)
    def _():
        acc_ref[...] = part_sum

    @pl.when(i > 0)
    def _():
        acc_ref[...] = acc_ref[...] + part_sum

    @pl.when(i == NBLK - 1)
    def _():
        m = acc_ref[...] / jnp.float32(N_NODES)
        v = jnp.sum(m * wv_ref[...], axis=1, keepdims=True) + bv_ref[...]
        out_ref[...] = jnp.tanh(v)


def _mlp(parts_p, xa_p, w1pp, e8s, b1p, w2t, b2, w3t, b3, wv, bv):
    return pl.pallas_call(
        _mlp_body,
        grid=(NBLK,),
        in_specs=[
            pl.BlockSpec((2, PBLK, 128), lambda i: (0, i, 0)),
            pl.BlockSpec((PBLK, 128), lambda i: (i, 0)),
            pl.BlockSpec((128, PWIDE), lambda i: (0, 0)),
            pl.BlockSpec((128, 128), lambda i: (0, 0)),
            pl.BlockSpec((1, PWIDE), lambda i: (0, 0)),
            pl.BlockSpec((HIDDEN, HIDDEN), lambda i: (0, 0)),
            pl.BlockSpec((1, HIDDEN), lambda i: (0, 0)),
            pl.BlockSpec((HIDDEN, HIDDEN), lambda i: (0, 0)),
            pl.BlockSpec((1, HIDDEN), lambda i: (0, 0)),
            pl.BlockSpec((1, HIDDEN), lambda i: (0, 0)),
            pl.BlockSpec((1, 1), lambda i: (0, 0)),
        ],
        out_specs=pl.BlockSpec((1, 1), lambda i: (0, 0)),
        out_shape=jax.ShapeDtypeStruct((1, 1), jnp.float32),
        scratch_shapes=[pltpu.VMEM((1, HIDDEN), jnp.float32)],
        compiler_params=pltpu.CompilerParams(
            dimension_semantics=("arbitrary",)),
    )(parts_p, xa_p, w1pp, e8s, b1p, w2t, b2, w3t, b3, wv, bv)


def kernel(x, edge_index, W1, b1, W2, b2, W3, b3, Wv, bv):
    ei3 = edge_index
    x3 = x.reshape(N_NODES // PACK, PACK, IN_DIM)
    x3 = jnp.concatenate(
        [x3,
         jnp.ones((N_NODES // PACK, PACK, 1), jnp.float32),
         jnp.zeros((N_NODES // PACK, PACK, XCOLS - IN_DIM - 1), jnp.float32)],
        axis=2)
    x3 = jnp.pad(x3, ((0, PROWS - N_NODES // PACK), (0, 0), (0, 0)))
    xa_p = x3.reshape(PROWS, 128)
    xaug = xa_p.reshape(AGG_ROWS, XCOLS)
    zeros_hbm = jnp.zeros((ZROWS, XCOLS), dtype=jnp.float32)

    parts = _scatter_parts(xaug, ei3, zeros_hbm)
    parts_p = parts.reshape(2, PROWS, 128)

    w1p = jnp.zeros((XCOLS, HIDDEN), dtype=jnp.float32).at[0:IN_DIM].set(W1.T)
    w1pp = jnp.zeros((128, PWIDE), dtype=jnp.float32)
    e8s = jnp.zeros((128, 128), dtype=jnp.float32)
    for j in range(PACK):
        w1pp = lax.dynamic_update_slice(w1pp, w1p, (j * XCOLS, j * HIDDEN))
        e8s = lax.dynamic_update_slice(
            e8s, jnp.ones((1, XCOLS), jnp.float32),
            (j * XCOLS + IN_DIM, j * XCOLS))
    b1p = jnp.tile(b1.reshape(1, -1), (1, PACK))

    out = _mlp(parts_p, xa_p, w1pp, e8s, b1p, W2.T,
               b2.reshape(1, -1), W3.T, b3.reshape(1, -1), Wv.reshape(1, -1),
               bv.reshape(1, 1))
    return jnp.squeeze(out)


# speedup vs baseline: 1.8049x; 1.0000x over previous
import functools

import jax
import jax.numpy as jnp
from jax import lax
from jax.experimental import pallas as pl
from jax.experimental.pallas import tpu as pltpu
from jax.experimental.pallas import tpu_sc as plsc

N_NODES = 100000
N_EDGES = 6400000
IN_DIM = 8
HIDDEN = 128
XCOLS = 16

NUM_WORKERS = 32
ROW = 128
ROWS_PER_ITER = 5
EDGE_ROWS = N_EDGES // ROW
N_CHUNKS = EDGE_ROWS // ROWS_PER_ITER
CHUNKS_MAIN = -(-N_CHUNKS // NUM_WORKERS)
CHUNKS_LAST = N_CHUNKS - 31 * CHUNKS_MAIN
AGG_ROWS = 102400
ZROWS = AGG_ROWS // 16


def _sc_body(xaug_hbm, ei_hbm, zeros_hbm, out_hbm,
             idx_v0, idx_v1, rows_v0, rows_v1, agg_sh,
             gsem0, gsem1, ssem0, ssem1, isem0, isem1):
    c = lax.axis_index("c")
    s = lax.axis_index("s")
    w = c * 16 + s

    pltpu.sync_copy(zeros_hbm, agg_sh.at[pl.ds(s * ZROWS, ZROWS)])
    plsc.subcore_barrier()

    base_chunk = w * CHUNKS_MAIN
    n = jnp.where(w == NUM_WORKERS - 1, CHUNKS_LAST, CHUNKS_MAIN)

    def fire_idx(i, idx_v, isem):
        r0 = (base_chunk + i) * ROWS_PER_ITER
        pltpu.async_copy(ei_hbm.at[:, pl.ds(r0, ROWS_PER_ITER)], idx_v, isem)

    def wait_idx(i, idx_v, isem):
        r0 = (base_chunk + i) * ROWS_PER_ITER
        pltpu.make_async_copy(ei_hbm.at[:, pl.ds(r0, ROWS_PER_ITER)],
                              idx_v, isem).wait()

    def fire_gathers(idx_v, rows_v, gsem):
        for j in range(ROWS_PER_ITER):
            pltpu.async_copy(xaug_hbm.at[idx_v.at[1].at[j]],
                             rows_v.at[j], gsem)

    def drain_gathers(idx_v, rows_v, gsem):
        for j in range(ROWS_PER_ITER):
            pltpu.make_async_copy(xaug_hbm.at[idx_v.at[1].at[j]],
                                  rows_v.at[j], gsem).wait()

    def fire_scatters(idx_v, rows_v, ssem):
        for j in range(ROWS_PER_ITER):
            pltpu.async_copy(rows_v.at[j],
                             agg_sh.at[idx_v.at[0].at[j]], ssem, add=True)

    def drain_scatters(idx_v, rows_v, ssem):
        for j in range(ROWS_PER_ITER):
            pltpu.make_async_copy(rows_v.at[j],
                                  agg_sh.at[idx_v.at[0].at[j]], ssem).wait()

    bufs = ((idx_v0, rows_v0, gsem0, ssem0, isem0),
            (idx_v1, rows_v1, gsem1, ssem1, isem1))

    fire_idx(0, idx_v0, isem0)
    wait_idx(0, idx_v0, isem0)
    fire_gathers(idx_v0, rows_v0, gsem0)

    def stage(i, cur, nxt):
        idx_c, rows_c, gsem_c, ssem_c, isem_c = cur
        idx_n, rows_n, gsem_n, ssem_n, isem_n = nxt

        @pl.when(i >= 1)
        def _():
            drain_scatters(idx_n, rows_n, ssem_n)

        @pl.when(i + 1 < n)
        def _():
            fire_idx(i + 1, idx_n, isem_n)

        drain_gathers(idx_c, rows_c, gsem_c)

        @pl.when(i + 1 < n)
        def _():
            wait_idx(i + 1, idx_n, isem_n)
            fire_gathers(idx_n, rows_n, gsem_n)

        fire_scatters(idx_c, rows_c, ssem_c)

    @pl.loop(0, n)
    def _edge_iter(i):
        even = (i % 2) == 0

        @pl.when(even)
        def _():
            stage(i, bufs[0], bufs[1])

        @pl.when(jnp.logical_not(even))
        def _():
            stage(i, bufs[1], bufs[0])

    last_even = ((n - 1) % 2) == 0

    @pl.when(last_even)
    def _():
        drain_scatters(idx_v0, rows_v0, ssem0)

    @pl.when(jnp.logical_not(last_even))
    def _():
        drain_scatters(idx_v1, rows_v1, ssem1)

    plsc.subcore_barrier()
    pltpu.sync_copy(agg_sh.at[pl.ds(s * ZROWS, ZROWS)],
                    out_hbm.at[c].at[pl.ds(s * ZROWS, ZROWS)])


def _scatter_parts(xaug, ei3, zeros_hbm):
    mesh = plsc.VectorSubcoreMesh(core_axis_name="c", subcore_axis_name="s")
    f = pl.kernel(
        _sc_body,
        out_type=jax.ShapeDtypeStruct((2, AGG_ROWS, XCOLS), jnp.float32),
        mesh=mesh,
        scratch_types=[
            pltpu.VMEM((2, ROWS_PER_ITER, ROW), jnp.int32),
            pltpu.VMEM((2, ROWS_PER_ITER, ROW), jnp.int32),
            pltpu.VMEM((ROWS_PER_ITER, ROW, XCOLS), jnp.float32),
            pltpu.VMEM((ROWS_PER_ITER, ROW, XCOLS), jnp.float32),
            pltpu.VMEM_SHARED((AGG_ROWS, XCOLS), jnp.float32),
            pltpu.SemaphoreType.DMA,
            pltpu.SemaphoreType.DMA,
            pltpu.SemaphoreType.DMA,
            pltpu.SemaphoreType.DMA,
            pltpu.SemaphoreType.DMA,
            pltpu.SemaphoreType.DMA,
        ],
        compiler_params=pltpu.CompilerParams(use_tc_tiling_on_sc=False),
    )
    return f(xaug, ei3, zeros_hbm)


PACK = 128 // XCOLS
PROWS = AGG_ROWS // PACK
NBLK = 50
PBLK = PROWS // NBLK
BLK = PBLK * PACK
PWIDE = PACK * HIDDEN


def _mlp_body(parts_ref, xa_ref, w1pp_ref, e8s_ref, b1p_ref, w2t_ref,
              b2_ref, w3t_ref, b3_ref, wv_ref, bv_ref, out_ref, acc_ref):
    i = pl.program_id(0)
    p2 = parts_ref[0] + parts_ref[1]
    dims = (((1,), (0,)), ((), ()))
    degp = jnp.maximum(lax.dot_general(p2, e8s_ref[...], dims), 1.0)
    q = xa_ref[...] + p2 / degp
    h1p = jnp.maximum(lax.dot_general(q, w1pp_ref[...], dims)
                      + b1p_ref[...], 0.0)
    h = h1p.reshape(BLK, HIDDEN)
    h = jnp.maximum(lax.dot_general(h, w2t_ref[...], dims)
                    + b2_ref[...], 0.0)
    h = jnp.maximum(lax.dot_general(h, w3t_ref[...], dims)
                    + b3_ref[...], 0.0)
    node = lax.broadcasted_iota(jnp.int32, (BLK, HIDDEN), 0) + i * BLK
    h = jnp.where(node < N_NODES, h, 0.0)
    part_sum = jnp.sum(h, axis=0, keepdims=True)

    @pl.when(i == 0)
    def _():
        acc_ref[...] = part_sum

    @pl.when(i > 0)
    def _():
        acc_ref[...] = acc_ref[...] + part_sum

    @pl.when(i == NBLK - 1)
    def _():
        m = acc_ref[...] / jnp.float32(N_NODES)
        v = jnp.sum(m * wv_ref[...], axis=1, keepdims=True) + bv_ref[...]
        out_ref[...] = jnp.tanh(v)


def _mlp(parts_p, xa_p, w1pp, e8s, b1p, w2t, b2, w3t, b3, wv, bv):
    return pl.pallas_call(
        _mlp_body,
        grid=(NBLK,),
        in_specs=[
            pl.BlockSpec((2, PBLK, 128), lambda i: (0, i, 0)),
            pl.BlockSpec((PBLK, 128), lambda i: (i, 0)),
            pl.BlockSpec((128, PWIDE), lambda i: (0, 0)),
            pl.BlockSpec((128, 128), lambda i: (0, 0)),
            pl.BlockSpec((1, PWIDE), lambda i: (0, 0)),
            pl.BlockSpec((HIDDEN, HIDDEN), lambda i: (0, 0)),
            pl.BlockSpec((1, HIDDEN), lambda i: (0, 0)),
            pl.BlockSpec((HIDDEN, HIDDEN), lambda i: (0, 0)),
            pl.BlockSpec((1, HIDDEN), lambda i: (0, 0)),
            pl.BlockSpec((1, HIDDEN), lambda i: (0, 0)),
            pl.BlockSpec((1, 1), lambda i: (0, 0)),
        ],
        out_specs=pl.BlockSpec((1, 1), lambda i: (0, 0)),
        out_shape=jax.ShapeDtypeStruct((1, 1), jnp.float32),
        scratch_shapes=[pltpu.VMEM((1, HIDDEN), jnp.float32)],
        compiler_params=pltpu.CompilerParams(
            dimension_semantics=("arbitrary",)),
    )(parts_p, xa_p, w1pp, e8s, b1p, w2t, b2, w3t, b3, wv, bv)


def kernel(x, edge_index, W1, b1, W2, b2, W3, b3, Wv, bv):
    ei3 = edge_index.reshape(2, EDGE_ROWS, ROW)
    x3 = x.reshape(N_NODES // PACK, PACK, IN_DIM)
    x3 = jnp.concatenate(
        [x3,
         jnp.ones((N_NODES // PACK, PACK, 1), jnp.float32),
         jnp.zeros((N_NODES // PACK, PACK, XCOLS - IN_DIM - 1), jnp.float32)],
        axis=2)
    x3 = jnp.pad(x3, ((0, PROWS - N_NODES // PACK), (0, 0), (0, 0)))
    xa_p = x3.reshape(PROWS, 128)
    xaug = xa_p.reshape(AGG_ROWS, XCOLS)
    zeros_hbm = jnp.zeros((ZROWS, XCOLS), dtype=jnp.float32)

    parts = _scatter_parts(xaug, ei3, zeros_hbm)
    parts_p = parts.reshape(2, PROWS, 128)

    w1p = jnp.zeros((XCOLS, HIDDEN), dtype=jnp.float32).at[0:IN_DIM].set(W1.T)
    w1pp = jnp.zeros((128, PWIDE), dtype=jnp.float32)
    e8s = jnp.zeros((128, 128), dtype=jnp.float32)
    for j in range(PACK):
        w1pp = lax.dynamic_update_slice(w1pp, w1p, (j * XCOLS, j * HIDDEN))
        e8s = lax.dynamic_update_slice(
            e8s, jnp.ones((1, XCOLS), jnp.float32),
            (j * XCOLS + IN_DIM, j * XCOLS))
    b1p = jnp.tile(b1.reshape(1, -1), (1, PACK))

    out = _mlp(parts_p, xa_p, w1pp, e8s, b1p, W2.T,
               b2.reshape(1, -1), W3.T, b3.reshape(1, -1), Wv.reshape(1, -1),
               bv.reshape(1, 1))
    return jnp.squeeze(out)
